# baseline (speedup 1.0000x reference)
import functools

import jax
import jax.numpy as jnp
from jax import lax
from jax.experimental import pallas as pl
from jax.experimental.pallas import tpu as pltpu
from jax.experimental.pallas import tpu_sc as plsc

D_MODEL = 128
N_BINS = 8
N_COMB = N_BINS ** 3
BATCH = 16384
HIST = 50
N_ROWS = BATCH * HIST
IDX_COLS = 128
CIDX_ROWS = N_ROWS // IDX_COLS

N_WORKERS = 32
PER_W_ROWS = N_ROWS // N_WORKERS
CHUNK = 512
G_SUB = CHUNK // IDX_COLS
CHUNKS_PER_SUPER = 2
SUPER_IDX_ROWS = CHUNKS_PER_SUPER * G_SUB
N_SUPER = PER_W_ROWS // (CHUNK * CHUNKS_PER_SUPER)


def _table_body(embS_ref, embU_ref, embF_ref, out_ref):
    c = lax.broadcasted_iota(jnp.int32, (N_COMB, N_BINS), 0)
    j = lax.broadcasted_iota(jnp.int32, (N_COMB, N_BINS), 1)
    ohS = jnp.where((c >> 6) == j, 1.0, 0.0)
    ohU = jnp.where(((c >> 3) & 7) == j, 1.0, 0.0)
    ohF = jnp.where((c & 7) == j, 1.0, 0.0)
    out_ref[...] = (
        jnp.dot(ohS, embS_ref[...], preferred_element_type=jnp.float32)
        + jnp.dot(ohU, embU_ref[...], preferred_element_type=jnp.float32)
        + jnp.dot(ohF, embF_ref[...], preferred_element_type=jnp.float32)
    )


def _build_table(embS, embU, embF):
    return pl.pallas_call(
        _table_body,
        out_shape=jax.ShapeDtypeStruct((N_COMB, D_MODEL), jnp.float32),
    )(embS, embU, embF)


def _cidx_body(s_ref, u_ref, f_ref, o_ref):
    o_ref[...] = s_ref[...] * 64 + u_ref[...] * 8 + f_ref[...]


def _combine_idx(binS, binU, binF):
    s = binS.reshape(CIDX_ROWS, IDX_COLS)
    u = binU.reshape(CIDX_ROWS, IDX_COLS)
    f = binF.reshape(CIDX_ROWS, IDX_COLS)
    grid = 8
    blk = CIDX_ROWS // grid
    spec = pl.BlockSpec((blk, IDX_COLS), lambda i: (i, 0))
    return pl.pallas_call(
        _cidx_body,
        grid=(grid,),
        in_specs=[spec, spec, spec],
        out_specs=spec,
        out_shape=jax.ShapeDtypeStruct((CIDX_ROWS, IDX_COLS), jnp.int32),
    )(s, u, f)


_mesh = plsc.VectorSubcoreMesh(core_axis_name="c", subcore_axis_name="s")


@functools.partial(
    pl.kernel,
    mesh=_mesh,
    out_type=jax.ShapeDtypeStruct((N_ROWS, D_MODEL), jnp.float32),
    scratch_types=[
        pltpu.VMEM((SUPER_IDX_ROWS, IDX_COLS), jnp.int32),
        pltpu.VMEM((CHUNK, D_MODEL), jnp.float32),
        pltpu.SemaphoreType.DMA,
    ],
)
def _sc_lookup(table_hbm, cidx_hbm, out_hbm, idx_v, rows_v, sem_g):
    wid = lax.axis_index("s") * 2 + lax.axis_index("c")
    row_base = wid * PER_W_ROWS
    idx_row_base = wid * PER_W_ROWS // IDX_COLS

    def body(i, carry):
        idx_off = pl.multiple_of(
            idx_row_base + i * SUPER_IDX_ROWS, 8)
        pltpu.sync_copy(
            cidx_hbm.at[pl.ds(idx_off, SUPER_IDX_ROWS)], idx_v)
        for c in range(CHUNKS_PER_SUPER):
            r0 = row_base + (i * CHUNKS_PER_SUPER + c) * CHUNK
            gathers = []
            for j in range(G_SUB):
                gathers.append(pltpu.async_copy(
                    table_hbm.at[idx_v.at[c * G_SUB + j]],
                    rows_v.at[pl.ds(j * IDX_COLS, IDX_COLS)],
                    sem_g,
                ))
            for cp in gathers:
                cp.wait()
            pltpu.sync_copy(rows_v, out_hbm.at[pl.ds(r0, CHUNK)])
        return carry

    lax.fori_loop(0, N_SUPER, body, 0)


def kernel(binS, binU, binF, embS, embU, embF):
    table = _build_table(embS, embU, embF)
    cidx = _combine_idx(binS, binU, binF)
    out = _sc_lookup(table, cidx)
    return out.reshape(BATCH, HIST, D_MODEL)


# speedup vs baseline: 1.9848x; 1.9848x over previous
import functools

import jax
import jax.numpy as jnp
from jax import lax
from jax.experimental import pallas as pl
from jax.experimental.pallas import tpu as pltpu
from jax.experimental.pallas import tpu_sc as plsc

D_MODEL = 128
N_BINS = 8
N_COMB = N_BINS ** 3
BATCH = 16384
HIST = 50
PAD_H = 56
LANES = 128
N_PAD_ROWS = BATCH * PAD_H

N_WORKERS = 32
B_PER_W = BATCH // N_WORKERS
NB = 16
CHUNK = NB * PAD_H
G_SUB = CHUNK // LANES
N_CHUNKS = B_PER_W // NB
SRC_IDX = NB * LANES
VOPS = (0, 16, 32, 40)


def _table_body(embS_ref, embU_ref, embF_ref, out_ref):
    c = lax.broadcasted_iota(jnp.int32, (N_COMB, N_BINS), 0)
    j = lax.broadcasted_iota(jnp.int32, (N_COMB, N_BINS), 1)
    ohS = jnp.where((c >> 6) == j, 1.0, 0.0)
    ohU = jnp.where(((c >> 3) & 7) == j, 1.0, 0.0)
    ohF = jnp.where((c & 7) == j, 1.0, 0.0)
    out_ref[...] = (
        jnp.dot(ohS, embS_ref[...], preferred_element_type=jnp.float32)
        + jnp.dot(ohU, embU_ref[...], preferred_element_type=jnp.float32)
        + jnp.dot(ohF, embF_ref[...], preferred_element_type=jnp.float32)
    )


def _build_table(embS, embU, embF):
    return pl.pallas_call(
        _table_body,
        out_shape=jax.ShapeDtypeStruct((N_COMB, D_MODEL), jnp.float32),
    )(embS, embU, embF)


def _cidx_body(s_ref, u_ref, f_ref, o_ref):
    blk = s_ref.shape[0]
    c = s_ref[...] * 64 + u_ref[...] * 8 + f_ref[...]
    i0 = lax.broadcasted_iota(jnp.int32, (blk, LANES - HIST), 0)
    i1 = lax.broadcasted_iota(jnp.int32, (blk, LANES - HIST), 1)
    filler = (i0 * 9 + i1 * 13) & (N_COMB - 1)
    o_ref[...] = jnp.concatenate([c, filler], axis=1)


def _combine_idx(binS, binU, binF):
    grid = 16
    blk = BATCH // grid
    in_spec = pl.BlockSpec((blk, HIST), lambda i: (i, 0))
    out_spec = pl.BlockSpec((blk, LANES), lambda i: (i, 0))
    return pl.pallas_call(
        _cidx_body,
        grid=(grid,),
        in_specs=[in_spec, in_spec, in_spec],
        out_specs=out_spec,
        out_shape=jax.ShapeDtypeStruct((BATCH, LANES), jnp.int32),
    )(binS, binU, binF)


_mesh = plsc.VectorSubcoreMesh(core_axis_name="c", subcore_axis_name="s")


@functools.partial(
    pl.kernel,
    mesh=_mesh,
    out_type=jax.ShapeDtypeStruct((N_PAD_ROWS, D_MODEL), jnp.float32),
    scratch_types=[
        pltpu.VMEM((SRC_IDX,), jnp.int32),
        pltpu.VMEM((CHUNK,), jnp.int32),
        pltpu.VMEM((CHUNK, D_MODEL), jnp.float32),
        pltpu.VMEM_SHARED((N_COMB, D_MODEL), jnp.float32),
        pltpu.SemaphoreType.DMA,
    ],
)
def _sc_lookup(table_hbm, cidx_hbm, out_hbm,
               src_v, idx_v, rows_v, table_sp, sem_g):
    wid = lax.axis_index("s") * 2 + lax.axis_index("c")
    row_base = wid * B_PER_W * PAD_H

    @pl.when(lax.axis_index("s") == 0)
    def _():
        pltpu.sync_copy(table_hbm, table_sp)
    plsc.subcore_barrier()

    def body(i, carry):
        b0 = wid * B_PER_W + i * NB
        src_off = pl.multiple_of(b0 * LANES, 8)
        pltpu.sync_copy(cidx_hbm.at[pl.ds(src_off, SRC_IDX)], src_v)
        for jb in range(NB):
            for o in VOPS:
                idx_v[pl.ds(jb * PAD_H + o, 16)] = (
                    src_v[pl.ds(jb * LANES + o, 16)])
        gathers = []
        for g in range(G_SUB):
            gathers.append(pltpu.async_copy(
                table_sp.at[idx_v.at[pl.ds(g * LANES, LANES)]],
                rows_v.at[pl.ds(g * LANES, LANES)],
                sem_g,
            ))
        for cp in gathers:
            cp.wait()
        pltpu.sync_copy(
            rows_v, out_hbm.at[pl.ds(row_base + i * CHUNK, CHUNK)])
        return carry

    lax.fori_loop(0, N_CHUNKS, body, 0)


def kernel(binS, binU, binF, embS, embU, embF):
    table = _build_table(embS, embU, embF)
    cidx = _combine_idx(binS, binU, binF).reshape(BATCH * LANES)
    out = _sc_lookup(table, cidx)
    return out.reshape(BATCH, PAD_H, D_MODEL)[:, :HIST, :]


# speedup vs baseline: 2.3276x; 1.1727x over previous
import functools

import jax
import jax.numpy as jnp
from jax import lax
from jax.experimental import pallas as pl
from jax.experimental.pallas import tpu as pltpu
from jax.experimental.pallas import tpu_sc as plsc

D_MODEL = 128
N_BINS = 8
N_COMB = N_BINS ** 3
BATCH = 16384
HIST = 50
PAD_H = 56
LANES = 128
N_PAD_ROWS = BATCH * PAD_H

N_WORKERS = 32
B_PER_W = BATCH // N_WORKERS
NB = 16
CHUNK = NB * PAD_H
G_SUB = CHUNK // LANES
N_CHUNKS = B_PER_W // NB
SRC_IDX = NB * LANES
VOPS = (0, 16, 32, 40)


def _table_body(embS_ref, embU_ref, embF_ref, out_ref):
    c = lax.broadcasted_iota(jnp.int32, (N_COMB, N_BINS), 0)
    j = lax.broadcasted_iota(jnp.int32, (N_COMB, N_BINS), 1)
    ohS = jnp.where((c >> 6) == j, 1.0, 0.0)
    ohU = jnp.where(((c >> 3) & 7) == j, 1.0, 0.0)
    ohF = jnp.where((c & 7) == j, 1.0, 0.0)
    out_ref[...] = (
        jnp.dot(ohS, embS_ref[...], preferred_element_type=jnp.float32)
        + jnp.dot(ohU, embU_ref[...], preferred_element_type=jnp.float32)
        + jnp.dot(ohF, embF_ref[...], preferred_element_type=jnp.float32)
    )


def _build_table(embS, embU, embF):
    return pl.pallas_call(
        _table_body,
        out_shape=jax.ShapeDtypeStruct((N_COMB, D_MODEL), jnp.float32),
    )(embS, embU, embF)


def _cidx_body(s_ref, u_ref, f_ref, o_ref):
    blk = s_ref.shape[0]
    c = s_ref[...] * 64 + u_ref[...] * 8 + f_ref[...]
    i0 = lax.broadcasted_iota(jnp.int32, (blk, LANES - HIST), 0)
    i1 = lax.broadcasted_iota(jnp.int32, (blk, LANES - HIST), 1)
    filler = (i0 * 9 + i1 * 13) & (N_COMB - 1)
    o_ref[...] = jnp.concatenate([c, filler], axis=1)


def _combine_idx(binS, binU, binF):
    grid = 16
    blk = BATCH // grid
    in_spec = pl.BlockSpec((blk, HIST), lambda i: (i, 0))
    out_spec = pl.BlockSpec((blk, LANES), lambda i: (i, 0))
    return pl.pallas_call(
        _cidx_body,
        grid=(grid,),
        in_specs=[in_spec, in_spec, in_spec],
        out_specs=out_spec,
        out_shape=jax.ShapeDtypeStruct((BATCH, LANES), jnp.int32),
    )(binS, binU, binF)


_mesh = plsc.VectorSubcoreMesh(core_axis_name="c", subcore_axis_name="s")


@functools.partial(
    pl.kernel,
    mesh=_mesh,
    out_type=jax.ShapeDtypeStruct((BATCH, HIST, D_MODEL), jnp.float32),
    scratch_types=[
        pltpu.VMEM((SRC_IDX,), jnp.int32),
        pltpu.VMEM((CHUNK,), jnp.int32),
        pltpu.VMEM((NB, HIST, D_MODEL), jnp.float32),
        pltpu.VMEM_SHARED((N_COMB, D_MODEL), jnp.float32),
        pltpu.SemaphoreType.DMA,
    ],
)
def _sc_lookup(table_hbm, cidx_hbm, out_hbm,
               src_v, idx_v, rows_v, table_sp, sem_g):
    wid = lax.axis_index("s") * 2 + lax.axis_index("c")

    @pl.when(lax.axis_index("s") == 0)
    def _():
        pltpu.sync_copy(table_hbm, table_sp)
    plsc.subcore_barrier()

    def body(i, carry):
        b0 = wid * B_PER_W + i * NB
        src_off = pl.multiple_of(b0 * LANES, 8)
        pltpu.sync_copy(cidx_hbm.at[pl.ds(src_off, SRC_IDX)], src_v)
        for jb in range(NB):
            for o in VOPS:
                idx_v[pl.ds(jb * PAD_H + o, 16)] = (
                    src_v[pl.ds(jb * LANES + o, 16)])
        gathers = []
        for jb in range(NB):
            gathers.append(pltpu.async_copy(
                table_sp.at[idx_v.at[pl.ds(jb * PAD_H, HIST)]],
                rows_v.at[jb],
                sem_g,
            ))
        for cp in gathers:
            cp.wait()
        pltpu.sync_copy(rows_v, out_hbm.at[pl.ds(b0, NB)])
        return carry

    lax.fori_loop(0, N_CHUNKS, body, 0)


def kernel(binS, binU, binF, embS, embU, embF):
    table = _build_table(embS, embU, embF)
    cidx = _combine_idx(binS, binU, binF).reshape(BATCH * LANES)
    return _sc_lookup(table, cidx)


# speedup vs baseline: 2.3377x; 1.0043x over previous
import functools

import jax
import jax.numpy as jnp
from jax import lax
from jax.experimental import pallas as pl
from jax.experimental.pallas import tpu as pltpu
from jax.experimental.pallas import tpu_sc as plsc

D_MODEL = 128
N_BINS = 8
N_COMB = N_BINS ** 3
BATCH = 16384
HIST = 50
PAD_H = 56
LANES = 128
N_PAD_ROWS = BATCH * PAD_H

N_WORKERS = 32
B_PER_W = BATCH // N_WORKERS
NB = 16
CHUNK = NB * PAD_H
G_SUB = CHUNK // LANES
N_CHUNKS = B_PER_W // NB
SRC_IDX = NB * LANES
VOPS = (0, 16, 32, 40)


def _table_body(embS_ref, embU_ref, embF_ref, out_ref):
    c = lax.broadcasted_iota(jnp.int32, (N_COMB, N_BINS), 0)
    j = lax.broadcasted_iota(jnp.int32, (N_COMB, N_BINS), 1)
    ohS = jnp.where((c >> 6) == j, 1.0, 0.0)
    ohU = jnp.where(((c >> 3) & 7) == j, 1.0, 0.0)
    ohF = jnp.where((c & 7) == j, 1.0, 0.0)
    out_ref[...] = (
        jnp.dot(ohS, embS_ref[...], preferred_element_type=jnp.float32)
        + jnp.dot(ohU, embU_ref[...], preferred_element_type=jnp.float32)
        + jnp.dot(ohF, embF_ref[...], preferred_element_type=jnp.float32)
    )


def _build_table(embS, embU, embF):
    return pl.pallas_call(
        _table_body,
        out_shape=jax.ShapeDtypeStruct((N_COMB, D_MODEL), jnp.float32),
    )(embS, embU, embF)


def _cidx_body(s_ref, u_ref, f_ref, o_ref):
    blk = s_ref.shape[0]
    c = s_ref[...] * 64 + u_ref[...] * 8 + f_ref[...]
    i0 = lax.broadcasted_iota(jnp.int32, (blk, LANES - HIST), 0)
    i1 = lax.broadcasted_iota(jnp.int32, (blk, LANES - HIST), 1)
    filler = (i0 * 9 + i1 * 13) & (N_COMB - 1)
    o_ref[...] = jnp.concatenate([c, filler], axis=1)


def _combine_idx(binS, binU, binF):
    grid = 16
    blk = BATCH // grid
    in_spec = pl.BlockSpec((blk, HIST), lambda i: (i, 0))
    out_spec = pl.BlockSpec((blk, LANES), lambda i: (i, 0))
    return pl.pallas_call(
        _cidx_body,
        grid=(grid,),
        in_specs=[in_spec, in_spec, in_spec],
        out_specs=out_spec,
        out_shape=jax.ShapeDtypeStruct((BATCH, LANES), jnp.int32),
    )(binS, binU, binF)


_mesh = plsc.VectorSubcoreMesh(core_axis_name="c", subcore_axis_name="s")


@functools.partial(
    pl.kernel,
    mesh=_mesh,
    compiler_params=pltpu.CompilerParams(use_tc_tiling_on_sc=True),
    out_type=jax.ShapeDtypeStruct((BATCH, HIST, D_MODEL), jnp.float32),
    scratch_types=[
        pltpu.VMEM((SRC_IDX,), jnp.int32),
        pltpu.VMEM((CHUNK,), jnp.int32),
        pltpu.VMEM((NB, HIST, D_MODEL), jnp.float32),
        pltpu.VMEM_SHARED((N_COMB, D_MODEL), jnp.float32),
        pltpu.SemaphoreType.DMA,
    ],
)
def _sc_lookup(table_hbm, cidx_hbm, out_hbm,
               src_v, idx_v, rows_v, table_sp, sem_g):
    wid = lax.axis_index("s") * 2 + lax.axis_index("c")

    @pl.when(lax.axis_index("s") == 0)
    def _():
        pltpu.sync_copy(table_hbm, table_sp)
    plsc.subcore_barrier()

    def body(i, carry):
        b0 = wid * B_PER_W + i * NB
        src_off = pl.multiple_of(b0 * LANES, 8)
        pltpu.sync_copy(cidx_hbm.at[pl.ds(src_off, SRC_IDX)], src_v)
        for jb in range(NB):
            for o in VOPS:
                idx_v[pl.ds(jb * PAD_H + o, 16)] = (
                    src_v[pl.ds(jb * LANES + o, 16)])
        gathers = []
        for jb in range(NB):
            gathers.append(pltpu.async_copy(
                table_sp.at[idx_v.at[pl.ds(jb * PAD_H, HIST)]],
                rows_v.at[jb],
                sem_g,
            ))
        for cp in gathers:
            cp.wait()
        pltpu.sync_copy(rows_v, out_hbm.at[pl.ds(b0, NB)])
        return carry

    lax.fori_loop(0, N_CHUNKS, body, 0)


def kernel(binS, binU, binF, embS, embU, embF):
    table = _build_table(embS, embU, embF)
    cidx = _combine_idx(binS, binU, binF).reshape(BATCH * LANES)
    return _sc_lookup(table, cidx)


# speedup vs baseline: 2.4670x; 1.0553x over previous
import functools

import jax
import jax.numpy as jnp
from jax import lax
from jax.experimental import pallas as pl
from jax.experimental.pallas import tpu as pltpu
from jax.experimental.pallas import tpu_sc as plsc

D_MODEL = 128
N_BINS = 8
N_COMB = N_BINS ** 3
BATCH = 16384
HIST = 50
PAD_H = 56
LANES = 128
N_PAD_ROWS = BATCH * PAD_H

N_WORKERS = 32
B_PER_W = BATCH // N_WORKERS
NB = 16
CHUNK = NB * PAD_H
G_SUB = CHUNK // LANES
N_CHUNKS = B_PER_W // NB
SRC_IDX = NB * LANES
VOPS = (0, 16, 32, 40)


def _table_body(embS_ref, embU_ref, embF_ref, out_ref):
    c = lax.broadcasted_iota(jnp.int32, (N_COMB, N_BINS), 0)
    j = lax.broadcasted_iota(jnp.int32, (N_COMB, N_BINS), 1)
    ohS = jnp.where((c >> 6) == j, 1.0, 0.0)
    ohU = jnp.where(((c >> 3) & 7) == j, 1.0, 0.0)
    ohF = jnp.where((c & 7) == j, 1.0, 0.0)
    out_ref[...] = (
        jnp.dot(ohS, embS_ref[...], preferred_element_type=jnp.float32)
        + jnp.dot(ohU, embU_ref[...], preferred_element_type=jnp.float32)
        + jnp.dot(ohF, embF_ref[...], preferred_element_type=jnp.float32)
    )


def _build_table(embS, embU, embF):
    return pl.pallas_call(
        _table_body,
        out_shape=jax.ShapeDtypeStruct((N_COMB, D_MODEL), jnp.float32),
    )(embS, embU, embF)


def _cidx_body(s_ref, u_ref, f_ref, o_ref):
    blk = s_ref.shape[0]
    c = s_ref[...] * 64 + u_ref[...] * 8 + f_ref[...]
    i0 = lax.broadcasted_iota(jnp.int32, (blk, LANES - HIST), 0)
    i1 = lax.broadcasted_iota(jnp.int32, (blk, LANES - HIST), 1)
    filler = (i0 * 9 + i1 * 13) & (N_COMB - 1)
    o_ref[...] = jnp.concatenate([c, filler], axis=1)


def _combine_idx(binS, binU, binF):
    grid = 16
    blk = BATCH // grid
    in_spec = pl.BlockSpec((blk, HIST), lambda i: (i, 0))
    out_spec = pl.BlockSpec((blk, LANES), lambda i: (i, 0))
    return pl.pallas_call(
        _cidx_body,
        grid=(grid,),
        in_specs=[in_spec, in_spec, in_spec],
        out_specs=out_spec,
        out_shape=jax.ShapeDtypeStruct((BATCH, LANES), jnp.int32),
    )(binS, binU, binF)


_mesh = plsc.VectorSubcoreMesh(core_axis_name="c", subcore_axis_name="s")


NB_H = NB // 2


@functools.partial(
    pl.kernel,
    mesh=_mesh,
    out_type=jax.ShapeDtypeStruct((BATCH, HIST, D_MODEL), jnp.float32),
    scratch_types=[
        pltpu.VMEM((SRC_IDX,), jnp.int32),
        pltpu.VMEM((CHUNK,), jnp.int32),
        pltpu.VMEM((NB, HIST, D_MODEL), jnp.float32),
        pltpu.VMEM_SHARED((N_COMB, D_MODEL), jnp.float32),
        pltpu.SemaphoreType.DMA,
        pltpu.SemaphoreType.DMA,
        pltpu.SemaphoreType.DMA,
        pltpu.SemaphoreType.DMA,
    ],
)
def _sc_lookup(table_hbm, cidx_hbm, out_hbm,
               src_v, idx_v, rows_v, table_sp,
               sem_g0, sem_g1, sem_s0, sem_s1):
    wid = lax.axis_index("s") * 2 + lax.axis_index("c")

    @pl.when(lax.axis_index("s") == 0)
    def _():
        pltpu.sync_copy(table_hbm, table_sp)
    plsc.subcore_barrier()

    def fire_gathers(half, sem):
        for jb in range(half * NB_H, (half + 1) * NB_H):
            pltpu.async_copy(
                table_sp.at[idx_v.at[pl.ds(jb * PAD_H, HIST)]],
                rows_v.at[jb],
                sem,
            )

    def wait_gathers(half, sem):
        for jb in range(half * NB_H, (half + 1) * NB_H):
            pltpu.make_async_copy(
                table_sp.at[idx_v.at[pl.ds(jb * PAD_H, HIST)]],
                rows_v.at[jb],
                sem,
            ).wait()

    def scatter_half(b0, half, sem):
        return pltpu.async_copy(
            rows_v.at[pl.ds(half * NB_H, NB_H)],
            out_hbm.at[pl.ds(b0 + half * NB_H, NB_H)],
            sem,
        )

    def wait_scatter_half(b0, half, sem):
        pltpu.make_async_copy(
            rows_v.at[pl.ds(half * NB_H, NB_H)],
            out_hbm.at[pl.ds(b0 + half * NB_H, NB_H)],
            sem,
        ).wait()

    def body(i, carry):
        b0 = wid * B_PER_W + i * NB
        src_off = pl.multiple_of(b0 * LANES, 8)
        pltpu.sync_copy(cidx_hbm.at[pl.ds(src_off, SRC_IDX)], src_v)
        for jb in range(NB):
            for o in VOPS:
                idx_v[pl.ds(jb * PAD_H + o, 16)] = (
                    src_v[pl.ds(jb * LANES + o, 16)])
        @pl.when(i > 0)
        def _():
            wait_scatter_half(b0 - NB, 0, sem_s0)
        fire_gathers(0, sem_g0)

        @pl.when(i > 0)
        def _():
            wait_scatter_half(b0 - NB, 1, sem_s1)
        fire_gathers(1, sem_g1)

        wait_gathers(0, sem_g0)
        scatter_half(b0, 0, sem_s0)
        wait_gathers(1, sem_g1)
        scatter_half(b0, 1, sem_s1)
        return carry

    lax.fori_loop(0, N_CHUNKS, body, 0)
    last_b0 = wid * B_PER_W + (N_CHUNKS - 1) * NB
    wait_scatter_half(last_b0, 0, sem_s0)
    wait_scatter_half(last_b0, 1, sem_s1)


def kernel(binS, binU, binF, embS, embU, embF):
    table = _build_table(embS, embU, embF)
    cidx = _combine_idx(binS, binU, binF).reshape(BATCH * LANES)
    return _sc_lookup(table, cidx)


# speedup vs baseline: 2.4727x; 1.0023x over previous
import functools

import jax
import jax.numpy as jnp
from jax import lax
from jax.experimental import pallas as pl
from jax.experimental.pallas import tpu as pltpu
from jax.experimental.pallas import tpu_sc as plsc

D_MODEL = 128
N_BINS = 8
N_COMB = N_BINS ** 3
BATCH = 16384
HIST = 50
PAD_H = 56
LANES = 128

N_WORKERS = 32
B_PER_W = BATCH // N_WORKERS
NB = 16
CHUNK = NB * PAD_H
N_CHUNKS = B_PER_W // NB
SRC_IDX = NB * LANES
VOPS = (0, 16, 32, 40)


def _table_body(embS_ref, embU_ref, embF_ref, out_ref):
    c = lax.broadcasted_iota(jnp.int32, (N_COMB, N_BINS), 0)
    j = lax.broadcasted_iota(jnp.int32, (N_COMB, N_BINS), 1)
    ohS = jnp.where((c >> 6) == j, 1.0, 0.0)
    ohU = jnp.where(((c >> 3) & 7) == j, 1.0, 0.0)
    ohF = jnp.where((c & 7) == j, 1.0, 0.0)
    out_ref[...] = (
        jnp.dot(ohS, embS_ref[...], preferred_element_type=jnp.float32)
        + jnp.dot(ohU, embU_ref[...], preferred_element_type=jnp.float32)
        + jnp.dot(ohF, embF_ref[...], preferred_element_type=jnp.float32)
    )


def _build_table(embS, embU, embF):
    return pl.pallas_call(
        _table_body,
        out_shape=jax.ShapeDtypeStruct((N_COMB, D_MODEL), jnp.float32),
    )(embS, embU, embF)


def _cidx_body(s_ref, u_ref, f_ref, o_ref):
    blk = s_ref.shape[0]
    c = s_ref[...] * 64 + u_ref[...] * 8 + f_ref[...]
    i0 = lax.broadcasted_iota(jnp.int32, (blk, LANES - HIST), 0)
    i1 = lax.broadcasted_iota(jnp.int32, (blk, LANES - HIST), 1)
    filler = (i0 * 9 + i1 * 13) & (N_COMB - 1)
    o_ref[...] = jnp.concatenate([c, filler], axis=1)


def _combine_idx(binS, binU, binF):
    grid = 16
    blk = BATCH // grid
    in_spec = pl.BlockSpec((blk, HIST), lambda i: (i, 0))
    out_spec = pl.BlockSpec((blk, LANES), lambda i: (i, 0))
    return pl.pallas_call(
        _cidx_body,
        grid=(grid,),
        in_specs=[in_spec, in_spec, in_spec],
        out_specs=out_spec,
        out_shape=jax.ShapeDtypeStruct((BATCH, LANES), jnp.int32),
    )(binS, binU, binF)


_mesh = plsc.VectorSubcoreMesh(core_axis_name="c", subcore_axis_name="s")


NB_H = NB // 2


@functools.partial(
    pl.kernel,
    mesh=_mesh,
    out_type=jax.ShapeDtypeStruct((BATCH, HIST, D_MODEL), jnp.float32),
    scratch_types=[
        pltpu.VMEM((SRC_IDX,), jnp.int32),
        pltpu.VMEM((CHUNK,), jnp.int32),
        pltpu.VMEM((NB, HIST, D_MODEL), jnp.float32),
        pltpu.VMEM_SHARED((N_COMB, D_MODEL), jnp.float32),
        pltpu.SemaphoreType.DMA,
        pltpu.SemaphoreType.DMA,
        pltpu.SemaphoreType.DMA,
        pltpu.SemaphoreType.DMA,
    ],
)
def _sc_lookup(table_hbm, cidx_hbm, out_hbm,
               src_v, idx_v, rows_v, table_sp,
               sem_g0, sem_g1, sem_s0, sem_s1):
    wid = lax.axis_index("s") * 2 + lax.axis_index("c")

    @pl.when(lax.axis_index("s") == 0)
    def _():
        pltpu.sync_copy(table_hbm, table_sp)
    plsc.subcore_barrier()

    def fire_gathers(half, sem):
        for jb in range(half * NB_H, (half + 1) * NB_H):
            pltpu.async_copy(
                table_sp.at[idx_v.at[pl.ds(jb * PAD_H, HIST)]],
                rows_v.at[jb],
                sem,
            )

    def wait_gathers(half, sem):
        for jb in range(half * NB_H, (half + 1) * NB_H):
            pltpu.make_async_copy(
                table_sp.at[idx_v.at[pl.ds(jb * PAD_H, HIST)]],
                rows_v.at[jb],
                sem,
            ).wait()

    def scatter_half(b0, half, sem):
        return pltpu.async_copy(
            rows_v.at[pl.ds(half * NB_H, NB_H)],
            out_hbm.at[pl.ds(b0 + half * NB_H, NB_H)],
            sem,
        )

    def wait_scatter_half(b0, half, sem):
        pltpu.make_async_copy(
            rows_v.at[pl.ds(half * NB_H, NB_H)],
            out_hbm.at[pl.ds(b0 + half * NB_H, NB_H)],
            sem,
        ).wait()

    def body(i, carry):
        b0 = wid * B_PER_W + i * NB
        src_off = pl.multiple_of(b0 * LANES, 8)
        pltpu.sync_copy(cidx_hbm.at[pl.ds(src_off, SRC_IDX)], src_v)
        for jb in range(NB):
            for o in VOPS:
                idx_v[pl.ds(jb * PAD_H + o, 16)] = (
                    src_v[pl.ds(jb * LANES + o, 16)])
        @pl.when(i > 0)
        def _():
            wait_scatter_half(b0 - NB, 0, sem_s0)
        fire_gathers(0, sem_g0)

        @pl.when(i > 0)
        def _():
            wait_scatter_half(b0 - NB, 1, sem_s1)
        fire_gathers(1, sem_g1)

        wait_gathers(0, sem_g0)
        scatter_half(b0, 0, sem_s0)
        wait_gathers(1, sem_g1)
        scatter_half(b0, 1, sem_s1)
        return carry

    lax.fori_loop(0, N_CHUNKS, body, 0)
    last_b0 = wid * B_PER_W + (N_CHUNKS - 1) * NB
    wait_scatter_half(last_b0, 0, sem_s0)
    wait_scatter_half(last_b0, 1, sem_s1)


def kernel(binS, binU, binF, embS, embU, embF):
    table = _build_table(embS, embU, embF)
    cidx = _combine_idx(binS, binU, binF).reshape(BATCH * LANES)
    return _sc_lookup(table, cidx)
